# baseline (speedup 1.0000x reference)
import functools

import jax
import jax.numpy as jnp
from jax import lax
from jax.experimental import pallas as pl
from jax.experimental.pallas import tpu as pltpu
from jax.experimental.pallas import tpu_sc as plsc

BATCH = 16384
EMBED_DIM = 64
SHARED_DIM = 16
NUM_WORKERS = 32
B_PER_W = BATCH // NUM_WORKERS
N_SUPER = 500000
CHUNK = 128
N_CHUNKS = B_PER_W // CHUNK


def _sc_kernel():
    mesh = plsc.VectorSubcoreMesh(core_axis_name="c", subcore_axis_name="s")

    @functools.partial(
        pl.kernel,
        out_type=jax.ShapeDtypeStruct((BATCH, EMBED_DIM), jnp.float32),
        mesh=mesh,
        scratch_types=[
            pltpu.VMEM((B_PER_W,), jnp.int32),
            pltpu.VMEM((N_CHUNKS, CHUNK), jnp.int32),
            pltpu.VMEM((2, CHUNK, 2 * EMBED_DIM), jnp.float32),
            pltpu.VMEM((B_PER_W, EMBED_DIM), jnp.float32),
            pltpu.VMEM((SHARED_DIM,), jnp.float32),
            pltpu.SemaphoreType.DMA,
            pltpu.SemaphoreType.DMA,
        ],
    )
    def k(x_hbm, w_hbm, sh_hbm, out_hbm, idx_v, sidx_v, rows_v,
          out_v, sh_v, sem0, sem1):
        sems = (sem0, sem1)
        wid = lax.axis_index("s") * 2 + lax.axis_index("c")
        base = wid * B_PER_W
        pltpu.sync_copy(x_hbm.at[pl.ds(base, B_PER_W)], idx_v)
        for j in range(B_PER_W // 16):
            v = idx_v[pl.ds(j * 16, 16)]
            sidx_v[j // 8, pl.ds((j % 8) * 16, 16)] = lax.shift_right_logical(v, 1)
        pltpu.sync_copy(sh_hbm.at[0], sh_v)
        sh = sh_v[...]

        copies = [None] * N_CHUNKS
        copies[0] = pltpu.async_copy(w_hbm.at[sidx_v.at[0]], rows_v.at[0],
                                     sems[0])
        for c in range(N_CHUNKS):
            if c + 1 < N_CHUNKS:
                copies[c + 1] = pltpu.async_copy(
                    w_hbm.at[sidx_v.at[c + 1]], rows_v.at[(c + 1) % 2],
                    sems[(c + 1) % 2])
            copies[c].wait()
            buf = c % 2

            def body(gr, carry, c=c, buf=buf):
                gbase = c * CHUNK + gr * 16
                par = idx_v[pl.ds(gbase, 16)] & 1
                for j in range(16):
                    g = gbase + j
                    r = gr * 16 + j
                    off = par[j] * EMBED_DIM
                    out_v[g, pl.ds(0, 16)] = sh
                    a = rows_v[buf, r, pl.ds(off + 16, 16)]
                    out_v[g, pl.ds(16, 16)] = a
                    b = rows_v[buf, r, pl.ds(off + 32, 16)]
                    out_v[g, pl.ds(32, 16)] = b
                    d = rows_v[buf, r, pl.ds(off + 48, 16)]
                    out_v[g, pl.ds(48, 16)] = d
                return carry

            lax.fori_loop(0, CHUNK // 16, body, 0)
        pltpu.sync_copy(out_v, out_hbm.at[pl.ds(base, B_PER_W)])

    return k


_k = _sc_kernel()


def kernel(X, W, shared_embed):
    W128 = W.reshape(N_SUPER, 2 * EMBED_DIM)
    return _k(X.astype(jnp.int32), W128, shared_embed)


# speedup vs baseline: 2.4918x; 2.4918x over previous
import functools

import jax
import jax.numpy as jnp
from jax import lax
from jax.experimental import pallas as pl
from jax.experimental.pallas import tpu as pltpu
from jax.experimental.pallas import tpu_sc as plsc

BATCH = 16384
EMBED_DIM = 64
SHARED_DIM = 16
NUM_WORKERS = 32
B_PER_W = BATCH // NUM_WORKERS
CHUNK = 32
N_CHUNKS = B_PER_W // CHUNK


def _sc_kernel():
    mesh = plsc.VectorSubcoreMesh(core_axis_name="c", subcore_axis_name="s")

    @functools.partial(
        pl.kernel,
        out_type=jax.ShapeDtypeStruct((BATCH, EMBED_DIM), jnp.float32),
        mesh=mesh,
        scratch_types=[
            pltpu.VMEM((B_PER_W + 16,), jnp.int32),
            pltpu.VMEM((2, CHUNK, EMBED_DIM), jnp.float32),
            pltpu.VMEM((SHARED_DIM,), jnp.float32),
            pltpu.SemaphoreType.DMA,
            pltpu.SemaphoreType.DMA,
            pltpu.SemaphoreType.DMA,
            pltpu.SemaphoreType.DMA,
        ],
    )
    def k(x_hbm, w_hbm, sh_hbm, out_hbm, idx_v, rows_v, sh_v,
          sem0, sem1, osem0, osem1):
        sems = (sem0, sem1)
        osems = (osem0, osem1)
        wid = lax.axis_index("s") * 2 + lax.axis_index("c")
        base = wid * B_PER_W
        pltpu.sync_copy(x_hbm.at[pl.ds(base, B_PER_W)],
                        idx_v.at[pl.ds(0, B_PER_W)])
        pltpu.sync_copy(sh_hbm.at[0], sh_v)
        sh = sh_v[...]

        def issue_chunk(c, buf):
            def issue(r, carry):
                g = c * CHUNK + r
                v = idx_v[pl.ds(g, 16)]
                i = v[0]
                t = lax.shift_right_logical(i, 3)
                a = i & 7
                pltpu.async_copy(w_hbm.at[t, a],
                                 rows_v.at[buf, r], sems[buf])
                return carry

            lax.fori_loop(0, CHUNK, issue, 0)

        def drain_chunk(buf):
            pltpu.make_async_copy(out_hbm.at[pl.ds(0, CHUNK)],
                                  rows_v.at[buf], sems[buf]).wait()

        out_pending = [False, False]
        issue_chunk(0, 0)
        for c in range(N_CHUNKS):
            buf = c % 2
            if c + 1 < N_CHUNKS:
                issue_chunk(c + 1, 1 - buf)
            drain_chunk(buf)
            if out_pending[buf]:
                pltpu.make_async_copy(
                    rows_v.at[buf],
                    out_hbm.at[pl.ds(0, CHUNK)], osems[buf]).wait()

            def patch(r, carry, buf=buf):
                rows_v[buf, r, pl.ds(0, SHARED_DIM)] = sh
                return carry

            lax.fori_loop(0, CHUNK, patch, 0, unroll=4)
            pltpu.async_copy(rows_v.at[buf],
                             out_hbm.at[pl.ds(base + c * CHUNK, CHUNK)],
                             osems[buf])
            out_pending[buf] = True
        for b in range(2):
            if out_pending[b]:
                pltpu.make_async_copy(
                    rows_v.at[b], out_hbm.at[pl.ds(0, CHUNK)],
                    osems[b]).wait()

    return k


_k = _sc_kernel()


def kernel(X, W, shared_embed):
    W8 = W.reshape(125000, 8, EMBED_DIM)
    return _k(X.astype(jnp.int32), W8, shared_embed)
